# baseline (speedup 1.0000x reference)
import functools

import jax
import jax.numpy as jnp
from jax.experimental import pallas as pl
from jax.experimental.pallas import tpu as pltpu


def _gate_kernel(inv_hw, x_ref, w1_ref, b1_ref, w2_ref, b2_ref, o_ref):
    x = x_ref[...]
    s = jnp.sum(x, axis=(1, 2))
    m = jnp.max(x, axis=(1, 2))
    pooled = jnp.stack([s * inv_hw, m], axis=0)
    hidden = jnp.maximum(
        jnp.dot(pooled, w1_ref[...],
                preferred_element_type=jnp.float32) + b1_ref[...], 0.0)
    att = jnp.dot(hidden, w2_ref[...],
                  preferred_element_type=jnp.float32) + b2_ref[...]
    scale = jax.nn.sigmoid(att[0:1, :] + att[1:2, :])
    o_ref[...] = x * scale.reshape(x.shape[0], 1, 1)


def _gate_half(xh, w1, b1r, b2r, w2):
    Bh, C, H, W = xh.shape
    R = w1.shape[1]
    x3 = xh.reshape(Bh * C, H, W)
    out = pl.pallas_call(
        functools.partial(_gate_kernel, 1.0 / float(H * W)),
        out_shape=jax.ShapeDtypeStruct((Bh * C, H, W), xh.dtype),
        grid=(Bh,),
        in_specs=[pl.BlockSpec((C, H, W), lambda b: (b, 0, 0)),
                  pl.BlockSpec((C, R), lambda b: (0, 0)),
                  pl.BlockSpec((1, R), lambda b: (0, 0)),
                  pl.BlockSpec((R, C), lambda b: (0, 0)),
                  pl.BlockSpec((1, C), lambda b: (0, 0))],
        out_specs=pl.BlockSpec((C, H, W), lambda b: (b, 0, 0)),
        compiler_params=pltpu.CompilerParams(
            dimension_semantics=("parallel",)),
    )(x3, w1, b1r, w2, b2r)
    return out.reshape(Bh, C, H, W)


def kernel(x, w1, b1, w2, b2):
    B, C, H, W = x.shape
    R = w1.shape[1]

    b1r = b1.reshape(1, R)
    b2r = b2.reshape(1, C)

    if B % 2:
        return _gate_half(x, w1, b1r, b2r, w2)
    Bh = B // 2
    halves = [_gate_half(x[i * Bh:(i + 1) * Bh], w1, b1r, b2r, w2)
              for i in range(2)]
    return jnp.concatenate(halves, axis=0)


# speedup vs baseline: 1.5338x; 1.5338x over previous
import functools

import jax
import jax.numpy as jnp
from jax.experimental import pallas as pl
from jax.experimental.pallas import tpu as pltpu


def _gate_kernel(inv_hw, x_ref, w1_ref, b1_ref, w2_ref, b2_ref, o_ref):
    x = x_ref[...]
    s = jnp.sum(x, axis=(1, 2))
    m = jnp.max(x, axis=(1, 2))
    pooled = jnp.stack([s * inv_hw, m], axis=0)
    hidden = jnp.maximum(
        jnp.dot(pooled, w1_ref[...],
                preferred_element_type=jnp.float32) + b1_ref[...], 0.0)
    att = jnp.dot(hidden, w2_ref[...],
                  preferred_element_type=jnp.float32) + b2_ref[...]
    scale = jax.nn.sigmoid(att[0:1, :] + att[1:2, :])
    o_ref[...] = x * scale.reshape(x.shape[0], 1, 1)


def kernel(x, w1, b1, w2, b2):
    B, C, H, W = x.shape
    R = w1.shape[1]

    x3 = x.reshape(B * C, H, W)
    b1r = b1.reshape(1, R)
    b2r = b2.reshape(1, C)

    out = pl.pallas_call(
        functools.partial(_gate_kernel, 1.0 / float(H * W)),
        out_shape=jax.ShapeDtypeStruct((B * C, H, W), x.dtype),
        grid=(B,),
        in_specs=[pl.BlockSpec((C, H, W), lambda b: (b, 0, 0)),
                  pl.BlockSpec((C, R), lambda b: (0, 0)),
                  pl.BlockSpec((1, R), lambda b: (0, 0)),
                  pl.BlockSpec((R, C), lambda b: (0, 0)),
                  pl.BlockSpec((1, C), lambda b: (0, 0))],
        out_specs=pl.BlockSpec((C, H, W), lambda b: (b, 0, 0)),
        compiler_params=pltpu.CompilerParams(
            dimension_semantics=("parallel",)),
    )(x3, w1, b1r, w2, b2r)

    return out.reshape(B, C, H, W)
